# baseline (speedup 1.0000x reference)
import jax
import jax.numpy as jnp
from jax import lax
from jax.experimental import pallas as pl

_NUM_TAGS = 64
_HIGH = 5.0
_LOW = -5.0

_B, _S, _K = 256, 40, 30
_KP, _LP = 32, 128
_NPAIR = _B * _S
_G = 512
_NSTEP = _NPAIR // _G


def _score_expand_body(labels_ref, out_ref):
    labs = labels_ref[...]
    labs_pad = jnp.concatenate([labs, labs[:, : _KP - _K]], axis=1)
    tags = lax.broadcasted_iota(jnp.int32, (_G, _KP, _LP), 2)
    out_ref[...] = jnp.where(
        tags == labs_pad[:, :, None], _HIGH, _LOW
    )


_score_expand = pl.pallas_call(
    _score_expand_body,
    out_shape=jax.ShapeDtypeStruct((_NPAIR, _KP, _LP), jnp.float32),
    grid=(_NSTEP,),
    in_specs=[pl.BlockSpec((_G, _K), lambda i: (i, 0))],
    out_specs=pl.BlockSpec((_G, _KP, _LP), lambda i: (i, 0, 0)),
)


def kernel(arg_labels):
    labels = arg_labels.astype(jnp.int32).reshape(_NPAIR, _K)
    out = _score_expand(labels)
    out = out[:, :_K, :_NUM_TAGS]
    return out.reshape(_B, _S, _K, _NUM_TAGS)
